# baseline (speedup 1.0000x reference)
import jax
import jax.numpy as jnp
from jax import lax
from jax.experimental import pallas as pl
from jax.experimental.pallas import tpu as pltpu

_TN = 4096
_GROUPS = 8
_EPS = 1e-5


def _fp_body(ptsT_ref, cT_ref, feats_ref, temb_ref, W_ref, b_ref, g_ref, bt_ref,
             out_ref, tout_ref, thi_ref, tlo_ref, ufeat_ref, s1_ref, s2_ref):
    M = cT_ref.shape[1]
    N = ufeat_ref.shape[1]
    OC = W_ref.shape[0]
    NT = N // _TN
    j = pl.program_id(1)

    @pl.when(j == 0)
    def _init():
        z = jnp.dot(W_ref[...], feats_ref[0],
                    preferred_element_type=jnp.float32) + b_ref[...]
        table = jnp.concatenate([z, temb_ref[0]], axis=0)
        thi = table.astype(jnp.bfloat16)
        thi_ref[...] = thi
        tlo_ref[...] = (table - thi.astype(jnp.float32)).astype(jnp.bfloat16)
        s1_ref[...] = jnp.zeros_like(s1_ref)
        s2_ref[...] = jnp.zeros_like(s2_ref)

    @pl.when(j < NT)
    def _tile():
        p = ptsT_ref[0]
        cT = cT_ref[0]
        pn2 = jnp.sum(p * p, axis=1, keepdims=True)
        cn2 = jnp.sum(cT * cT, axis=1).reshape(1, M)
        e = lax.dot_general(p, cT, (((1,), (1,)), ((), ())),
                            preferred_element_type=jnp.float32)
        score = (pn2 + cn2) - 2.0 * e
        idx = jnp.argmin(score, axis=1).reshape(_TN, 1)
        iota_m = lax.broadcasted_iota(jnp.int32, (_TN, M), 1)
        onehot = (iota_m == idx).astype(jnp.bfloat16)
        dn = (((1,), (1,)), ((), ()))
        u = lax.dot_general(thi_ref[...], onehot, dn,
                            preferred_element_type=jnp.float32)
        uf = u[0:OC]
        st = pl.multiple_of(j * _TN, _TN)
        ufeat_ref[:, pl.ds(st, _TN)] = uf
        tout_ref[0] = u[OC:]
        s1_ref[...] += jnp.sum(uf, axis=1, keepdims=True)
        s2_ref[...] += jnp.sum(uf * uf, axis=1, keepdims=True)

    @pl.when(j == NT)
    def _epilogue():
        cpg = OC // _GROUPS
        G = (lax.broadcasted_iota(jnp.int32, (_GROUPS, OC), 1) // cpg
             == lax.broadcasted_iota(jnp.int32, (_GROUPS, OC), 0)).astype(jnp.float32)
        GT = (lax.broadcasted_iota(jnp.int32, (OC, _GROUPS), 0) // cpg
              == lax.broadcasted_iota(jnp.int32, (OC, _GROUPS), 1)).astype(jnp.float32)
        denom = float(N * cpg)
        gmean = jnp.dot(G, s1_ref[...], preferred_element_type=jnp.float32) / denom
        gm2 = jnp.dot(G, s2_ref[...], preferred_element_type=jnp.float32) / denom
        gscale = lax.rsqrt(gm2 - gmean * gmean + _EPS)
        mean_c = jnp.dot(GT, gmean, preferred_element_type=jnp.float32)
        scale_c = jnp.dot(GT, gscale, preferred_element_type=jnp.float32)
        a = g_ref[...] * scale_c
        c0 = bt_ref[...] - g_ref[...] * scale_c * mean_c
        yn = ufeat_ref[...] * a + c0
        out_ref[0] = yn * jax.nn.sigmoid(yn)


def kernel(points_coords, centers_coords, centers_features, temb, W, b, gamma, beta):
    B, _, N = points_coords.shape
    M = centers_coords.shape[2]
    C = centers_features.shape[1]
    OC = W.shape[0]
    NT = N // _TN

    ptsT = jnp.transpose(points_coords, (0, 2, 1))
    cT = jnp.transpose(centers_coords, (0, 2, 1))
    b2 = b.reshape(OC, 1)
    g2 = gamma.reshape(OC, 1)
    bt2 = beta.reshape(OC, 1)

    last = NT - 1
    out, tout = pl.pallas_call(
        _fp_body,
        grid=(B, NT + 1),
        in_specs=[
            pl.BlockSpec((1, _TN, 3), lambda i, j: (i, jnp.minimum(j, last), 0)),
            pl.BlockSpec((1, M, 3), lambda i, j: (i, 0, 0)),
            pl.BlockSpec((1, C, M), lambda i, j: (i, 0, 0)),
            pl.BlockSpec((1, C, M), lambda i, j: (i, 0, 0)),
            pl.BlockSpec((OC, C), lambda i, j: (0, 0)),
            pl.BlockSpec((OC, 1), lambda i, j: (0, 0)),
            pl.BlockSpec((OC, 1), lambda i, j: (0, 0)),
            pl.BlockSpec((OC, 1), lambda i, j: (0, 0)),
        ],
        out_specs=[
            pl.BlockSpec((1, OC, N), lambda i, j: (i, 0, 0)),
            pl.BlockSpec((1, C, _TN), lambda i, j: (i, 0, jnp.minimum(j, last))),
        ],
        out_shape=[
            jax.ShapeDtypeStruct((B, OC, N), jnp.float32),
            jax.ShapeDtypeStruct((B, C, N), jnp.float32),
        ],
        scratch_shapes=[
            pltpu.VMEM((OC + C, M), jnp.bfloat16),
            pltpu.VMEM((OC + C, M), jnp.bfloat16),
            pltpu.VMEM((OC, N), jnp.float32),
            pltpu.VMEM((OC, 1), jnp.float32),
            pltpu.VMEM((OC, 1), jnp.float32),
        ],
    )(ptsT, cT, centers_features, temb, W, b2, g2, bt2)

    return (out, points_coords, tout)


# speedup vs baseline: 1.0043x; 1.0043x over previous
import jax
import jax.numpy as jnp
from jax import lax
from jax.experimental import pallas as pl
from jax.experimental.pallas import tpu as pltpu

_TN = 2048
_GROUPS = 8
_EPS = 1e-5


def _fp_body(ptsT_ref, cT_ref, feats_ref, temb_ref, W_ref, b_ref, g_ref, bt_ref,
             out_ref, tout_ref, thi_ref, tlo_ref, ufeat_ref, s1_ref, s2_ref):
    M = cT_ref.shape[1]
    N = ufeat_ref.shape[1]
    OC = W_ref.shape[0]
    NT = N // _TN
    j = pl.program_id(1)

    @pl.when(j == 0)
    def _init():
        z = jnp.dot(W_ref[...], feats_ref[0],
                    preferred_element_type=jnp.float32) + b_ref[...]
        table = jnp.concatenate([z, temb_ref[0]], axis=0)
        thi = table.astype(jnp.bfloat16)
        thi_ref[...] = thi
        tlo_ref[...] = (table - thi.astype(jnp.float32)).astype(jnp.bfloat16)
        s1_ref[...] = jnp.zeros_like(s1_ref)
        s2_ref[...] = jnp.zeros_like(s2_ref)

    @pl.when(j < NT)
    def _tile():
        p = ptsT_ref[0]
        cT = cT_ref[0]
        pn2 = jnp.sum(p * p, axis=1, keepdims=True)
        cn2 = jnp.sum(cT * cT, axis=1).reshape(1, M)
        e = lax.dot_general(p, cT, (((1,), (1,)), ((), ())),
                            preferred_element_type=jnp.float32)
        score = (pn2 + cn2) - 2.0 * e
        idx = jnp.argmin(score, axis=1).reshape(_TN, 1)
        iota_m = lax.broadcasted_iota(jnp.int32, (_TN, M), 1)
        onehot = (iota_m == idx).astype(jnp.bfloat16)
        dn = (((1,), (1,)), ((), ()))
        u = lax.dot_general(thi_ref[...], onehot, dn,
                            preferred_element_type=jnp.float32)
        uf = u[0:OC]
        st = pl.multiple_of(j * _TN, _TN)
        ufeat_ref[:, pl.ds(st, _TN)] = uf
        tout_ref[0] = u[OC:]
        s1_ref[...] += jnp.sum(uf, axis=1, keepdims=True)
        s2_ref[...] += jnp.sum(uf * uf, axis=1, keepdims=True)

    @pl.when(j == NT)
    def _epilogue():
        cpg = OC // _GROUPS
        G = (lax.broadcasted_iota(jnp.int32, (_GROUPS, OC), 1) // cpg
             == lax.broadcasted_iota(jnp.int32, (_GROUPS, OC), 0)).astype(jnp.float32)
        GT = (lax.broadcasted_iota(jnp.int32, (OC, _GROUPS), 0) // cpg
              == lax.broadcasted_iota(jnp.int32, (OC, _GROUPS), 1)).astype(jnp.float32)
        denom = float(N * cpg)
        gmean = jnp.dot(G, s1_ref[...], preferred_element_type=jnp.float32) / denom
        gm2 = jnp.dot(G, s2_ref[...], preferred_element_type=jnp.float32) / denom
        gscale = lax.rsqrt(gm2 - gmean * gmean + _EPS)
        mean_c = jnp.dot(GT, gmean, preferred_element_type=jnp.float32)
        scale_c = jnp.dot(GT, gscale, preferred_element_type=jnp.float32)
        a = g_ref[...] * scale_c
        c0 = bt_ref[...] - g_ref[...] * scale_c * mean_c
        yn = ufeat_ref[...] * a + c0
        out_ref[0] = yn * jax.nn.sigmoid(yn)


def kernel(points_coords, centers_coords, centers_features, temb, W, b, gamma, beta):
    B, _, N = points_coords.shape
    M = centers_coords.shape[2]
    C = centers_features.shape[1]
    OC = W.shape[0]
    NT = N // _TN

    ptsT = jnp.transpose(points_coords, (0, 2, 1))
    cT = jnp.transpose(centers_coords, (0, 2, 1))
    b2 = b.reshape(OC, 1)
    g2 = gamma.reshape(OC, 1)
    bt2 = beta.reshape(OC, 1)

    last = NT - 1
    out, tout = pl.pallas_call(
        _fp_body,
        grid=(B, NT + 1),
        in_specs=[
            pl.BlockSpec((1, _TN, 3), lambda i, j: (i, jnp.minimum(j, last), 0)),
            pl.BlockSpec((1, M, 3), lambda i, j: (i, 0, 0)),
            pl.BlockSpec((1, C, M), lambda i, j: (i, 0, 0)),
            pl.BlockSpec((1, C, M), lambda i, j: (i, 0, 0)),
            pl.BlockSpec((OC, C), lambda i, j: (0, 0)),
            pl.BlockSpec((OC, 1), lambda i, j: (0, 0)),
            pl.BlockSpec((OC, 1), lambda i, j: (0, 0)),
            pl.BlockSpec((OC, 1), lambda i, j: (0, 0)),
        ],
        out_specs=[
            pl.BlockSpec((1, OC, N), lambda i, j: (i, 0, 0)),
            pl.BlockSpec((1, C, _TN), lambda i, j: (i, 0, jnp.minimum(j, last))),
        ],
        out_shape=[
            jax.ShapeDtypeStruct((B, OC, N), jnp.float32),
            jax.ShapeDtypeStruct((B, C, N), jnp.float32),
        ],
        scratch_shapes=[
            pltpu.VMEM((OC + C, M), jnp.bfloat16),
            pltpu.VMEM((OC + C, M), jnp.bfloat16),
            pltpu.VMEM((OC, N), jnp.float32),
            pltpu.VMEM((OC, 1), jnp.float32),
            pltpu.VMEM((OC, 1), jnp.float32),
        ],
    )(ptsT, cT, centers_features, temb, W, b2, g2, bt2)

    return (out, points_coords, tout)


# speedup vs baseline: 1.0340x; 1.0296x over previous
import jax
import jax.numpy as jnp
from jax import lax
from jax.experimental import pallas as pl
from jax.experimental.pallas import tpu as pltpu

_TN = 2048
_GROUPS = 8
_EPS = 1e-5


def _fp_body(ptsT_ref, cT_ref, feats_ref, temb_ref, W_ref, b_ref, g_ref, bt_ref,
             out_ref, tout_ref, thi_ref, tlo_ref, ufeat_ref, s1_ref, s2_ref):
    M = cT_ref.shape[1]
    N = ufeat_ref.shape[1]
    OC = W_ref.shape[0]
    NT = N // _TN
    j = pl.program_id(1)

    @pl.when(j == 0)
    def _init():
        z = jnp.dot(W_ref[...], feats_ref[0],
                    preferred_element_type=jnp.float32) + b_ref[...]
        table = jnp.concatenate([z, temb_ref[0]], axis=0)
        thi = table.astype(jnp.bfloat16)
        thi_ref[...] = thi
        tlo_ref[...] = (table - thi.astype(jnp.float32)).astype(jnp.bfloat16)
        s1_ref[...] = jnp.zeros_like(s1_ref)
        s2_ref[...] = jnp.zeros_like(s2_ref)

    @pl.when(j < NT)
    def _tile():
        p = ptsT_ref[0]
        cT = cT_ref[0]
        pn2 = jnp.sum(p * p, axis=1, keepdims=True)
        cn2 = jnp.sum(cT * cT, axis=1).reshape(1, M)
        e = lax.dot_general(p, cT, (((1,), (1,)), ((), ())),
                            preferred_element_type=jnp.float32)
        score = (pn2 + cn2) - 2.0 * e
        idx = jnp.argmin(score, axis=1).reshape(_TN, 1)
        iota_m = lax.broadcasted_iota(jnp.int32, (_TN, M), 1)
        onehot = (iota_m == idx).astype(jnp.bfloat16)
        dn = (((1,), (1,)), ((), ()))
        u = lax.dot_general(thi_ref[...], onehot, dn,
                            preferred_element_type=jnp.float32)
        uf = u[0:OC]
        st = pl.multiple_of(j * _TN, _TN)
        ufeat_ref[:, pl.ds(st, _TN)] = uf
        tout_ref[0] = u[OC:]
        s1_ref[...] += jnp.sum(uf, axis=1, keepdims=True)
        s2_ref[...] += jnp.sum(uf * uf, axis=1, keepdims=True)

    @pl.when(j == NT - 1)
    def _epilogue():
        cpg = OC // _GROUPS
        G = (lax.broadcasted_iota(jnp.int32, (_GROUPS, OC), 1) // cpg
             == lax.broadcasted_iota(jnp.int32, (_GROUPS, OC), 0)).astype(jnp.float32)
        GT = (lax.broadcasted_iota(jnp.int32, (OC, _GROUPS), 0) // cpg
              == lax.broadcasted_iota(jnp.int32, (OC, _GROUPS), 1)).astype(jnp.float32)
        denom = float(N * cpg)
        gmean = jnp.dot(G, s1_ref[...], preferred_element_type=jnp.float32) / denom
        gm2 = jnp.dot(G, s2_ref[...], preferred_element_type=jnp.float32) / denom
        gscale = lax.rsqrt(gm2 - gmean * gmean + _EPS)
        mean_c = jnp.dot(GT, gmean, preferred_element_type=jnp.float32)
        scale_c = jnp.dot(GT, gscale, preferred_element_type=jnp.float32)
        a = g_ref[...] * scale_c
        c0 = bt_ref[...] - g_ref[...] * scale_c * mean_c
        yn = ufeat_ref[...] * a + c0
        out_ref[0] = yn * jax.nn.sigmoid(yn)


def kernel(points_coords, centers_coords, centers_features, temb, W, b, gamma, beta):
    B, _, N = points_coords.shape
    M = centers_coords.shape[2]
    C = centers_features.shape[1]
    OC = W.shape[0]
    NT = N // _TN

    ptsT = jnp.transpose(points_coords, (0, 2, 1))
    cT = jnp.transpose(centers_coords, (0, 2, 1))
    b2 = b.reshape(OC, 1)
    g2 = gamma.reshape(OC, 1)
    bt2 = beta.reshape(OC, 1)

    out, tout = pl.pallas_call(
        _fp_body,
        grid=(B, NT),
        in_specs=[
            pl.BlockSpec((1, _TN, 3), lambda i, j: (i, j, 0)),
            pl.BlockSpec((1, M, 3), lambda i, j: (i, 0, 0)),
            pl.BlockSpec((1, C, M), lambda i, j: (i, 0, 0)),
            pl.BlockSpec((1, C, M), lambda i, j: (i, 0, 0)),
            pl.BlockSpec((OC, C), lambda i, j: (0, 0)),
            pl.BlockSpec((OC, 1), lambda i, j: (0, 0)),
            pl.BlockSpec((OC, 1), lambda i, j: (0, 0)),
            pl.BlockSpec((OC, 1), lambda i, j: (0, 0)),
        ],
        out_specs=[
            pl.BlockSpec((1, OC, N), lambda i, j: (i, 0, 0)),
            pl.BlockSpec((1, C, _TN), lambda i, j: (i, 0, j)),
        ],
        out_shape=[
            jax.ShapeDtypeStruct((B, OC, N), jnp.float32),
            jax.ShapeDtypeStruct((B, C, N), jnp.float32),
        ],
        scratch_shapes=[
            pltpu.VMEM((OC + C, M), jnp.bfloat16),
            pltpu.VMEM((OC + C, M), jnp.bfloat16),
            pltpu.VMEM((OC, N), jnp.float32),
            pltpu.VMEM((OC, 1), jnp.float32),
            pltpu.VMEM((OC, 1), jnp.float32),
        ],
    )(ptsT, cT, centers_features, temb, W, b2, g2, bt2)

    return (out, points_coords, tout)


# speedup vs baseline: 1.0348x; 1.0008x over previous
import jax
import jax.numpy as jnp
from jax import lax
from jax.experimental import pallas as pl
from jax.experimental.pallas import tpu as pltpu

_TN = 2048
_GROUPS = 8
_EPS = 1e-5


def _fp_body(ptsT_ref, cT_ref, feats_ref, temb_ref, W_ref, b_ref, g_ref, bt_ref,
             out_ref, tout_ref, thi_ref, tlo_ref, ufeat_ref, s1_ref, s2_ref):
    M = cT_ref.shape[1]
    N = ufeat_ref.shape[1]
    OC = W_ref.shape[0]
    NT = N // _TN
    j = pl.program_id(1)

    @pl.when(j == 0)
    def _init():
        z = jnp.dot(W_ref[...], feats_ref[0],
                    preferred_element_type=jnp.float32) + b_ref[...]
        table = jnp.concatenate([z, temb_ref[0]], axis=0)
        thi = table.astype(jnp.bfloat16)
        thi_ref[...] = thi
        tlo_ref[...] = (table - thi.astype(jnp.float32)).astype(jnp.bfloat16)
        s1_ref[...] = jnp.zeros_like(s1_ref)
        s2_ref[...] = jnp.zeros_like(s2_ref)

    p = ptsT_ref[0]
    cT = cT_ref[0]
    pn2 = jnp.sum(p * p, axis=1, keepdims=True)
    cn2 = jnp.sum(cT * cT, axis=1).reshape(1, M)
    e = lax.dot_general(p, cT, (((1,), (1,)), ((), ())),
                        preferred_element_type=jnp.float32)
    score = (pn2 + cn2) - 2.0 * e
    idx = jnp.argmin(score, axis=1).reshape(_TN, 1)
    iota_m = lax.broadcasted_iota(jnp.int32, (_TN, M), 1)
    onehot = (iota_m == idx).astype(jnp.bfloat16)
    dn = (((1,), (1,)), ((), ()))
    u = lax.dot_general(thi_ref[...], onehot, dn,
                        preferred_element_type=jnp.float32)
    uf = u[0:OC]
    st = pl.multiple_of(j * _TN, _TN)
    ufeat_ref[:, pl.ds(st, _TN)] = uf
    tout_ref[0] = u[OC:]
    s1_ref[...] += jnp.sum(uf, axis=1, keepdims=True)
    s2_ref[...] += jnp.sum(uf * uf, axis=1, keepdims=True)

    @pl.when(j == NT - 1)
    def _epilogue():
        cpg = OC // _GROUPS
        G = (lax.broadcasted_iota(jnp.int32, (_GROUPS, OC), 1) // cpg
             == lax.broadcasted_iota(jnp.int32, (_GROUPS, OC), 0)).astype(jnp.float32)
        GT = (lax.broadcasted_iota(jnp.int32, (OC, _GROUPS), 0) // cpg
              == lax.broadcasted_iota(jnp.int32, (OC, _GROUPS), 1)).astype(jnp.float32)
        denom = float(N * cpg)
        gmean = jnp.dot(G, s1_ref[...], preferred_element_type=jnp.float32) / denom
        gm2 = jnp.dot(G, s2_ref[...], preferred_element_type=jnp.float32) / denom
        gscale = lax.rsqrt(gm2 - gmean * gmean + _EPS)
        mean_c = jnp.dot(GT, gmean, preferred_element_type=jnp.float32)
        scale_c = jnp.dot(GT, gscale, preferred_element_type=jnp.float32)
        a = g_ref[...] * scale_c
        c0 = bt_ref[...] - g_ref[...] * scale_c * mean_c
        yn = ufeat_ref[...] * a + c0
        out_ref[0] = yn * jax.nn.sigmoid(yn)


def kernel(points_coords, centers_coords, centers_features, temb, W, b, gamma, beta):
    B, _, N = points_coords.shape
    M = centers_coords.shape[2]
    C = centers_features.shape[1]
    OC = W.shape[0]
    NT = N // _TN

    ptsT = jnp.transpose(points_coords, (0, 2, 1))
    cT = jnp.transpose(centers_coords, (0, 2, 1))
    b2 = b.reshape(OC, 1)
    g2 = gamma.reshape(OC, 1)
    bt2 = beta.reshape(OC, 1)

    out, tout = pl.pallas_call(
        _fp_body,
        grid=(B, NT),
        in_specs=[
            pl.BlockSpec((1, _TN, 3), lambda i, j: (i, j, 0)),
            pl.BlockSpec((1, M, 3), lambda i, j: (i, 0, 0)),
            pl.BlockSpec((1, C, M), lambda i, j: (i, 0, 0)),
            pl.BlockSpec((1, C, M), lambda i, j: (i, 0, 0)),
            pl.BlockSpec((OC, C), lambda i, j: (0, 0)),
            pl.BlockSpec((OC, 1), lambda i, j: (0, 0)),
            pl.BlockSpec((OC, 1), lambda i, j: (0, 0)),
            pl.BlockSpec((OC, 1), lambda i, j: (0, 0)),
        ],
        out_specs=[
            pl.BlockSpec((1, OC, N), lambda i, j: (i, 0, 0)),
            pl.BlockSpec((1, C, _TN), lambda i, j: (i, 0, j)),
        ],
        out_shape=[
            jax.ShapeDtypeStruct((B, OC, N), jnp.float32),
            jax.ShapeDtypeStruct((B, C, N), jnp.float32),
        ],
        scratch_shapes=[
            pltpu.VMEM((OC + C, M), jnp.bfloat16),
            pltpu.VMEM((OC + C, M), jnp.bfloat16),
            pltpu.VMEM((OC, N), jnp.float32),
            pltpu.VMEM((OC, 1), jnp.float32),
            pltpu.VMEM((OC, 1), jnp.float32),
        ],
    )(ptsT, cT, centers_features, temb, W, b2, g2, bt2)

    return (out, points_coords, tout)
